# baseline (speedup 1.0000x reference)
import jax
import jax.numpy as jnp
from jax import lax
from jax.experimental import pallas as pl
from jax.experimental.pallas import tpu as pltpu
from jax.experimental.pallas import tpu_sc as plsc

D = 64
L = 16
SCALE = 8.0
NC, NS = 2, 16
NW = NC * NS
S = 200
G = 40
NBUF = 4


def _body(x_hbm, table_hbm, out_hbm, idx_all,
          gb0, gb1, gb2, gb3, ob0, ob1, ob2, ob3,
          gs0, gs1, gs2, gs3, os0, os1, os2, os3):
    gbuf = [gb0, gb1, gb2, gb3]
    obuf = [ob0, ob1, ob2, ob3]
    gsem = [gs0, gs1, gs2, gs3]
    osem = [os0, os1, os2, os3]

    wid = lax.axis_index("s") * NC + lax.axis_index("c")
    rows_per_w = x_hbm.shape[0] // NW
    base = wid * rows_per_w

    pltpu.sync_copy(x_hbm.at[pl.ds(base, rows_per_w)], idx_all)

    def gather_start(r, b):
        for h in range(S // G):
            pltpu.async_copy(
                table_hbm.at[idx_all.at[r, h]],
                gbuf[b].at[pl.ds(h * G, G)], gsem[b])

    def gather_wait(r, b):
        for h in range(S // G):
            pltpu.make_async_copy(
                table_hbm.at[idx_all.at[r, h]],
                gbuf[b].at[pl.ds(h * G, G)], gsem[b]).wait()

    def out_start(r, b):
        pltpu.async_copy(obuf[b], out_hbm.at[pl.ds(base + r, 1)], osem[b])

    def out_wait(r, b):
        pltpu.make_async_copy(obuf[b], out_hbm.at[pl.ds(base + r, 1)],
                              osem[b]).wait()

    def scale(b):
        gbr, obr = gbuf[b], obuf[b]

        @pl.loop(0, S, unroll=4)
        def _(i):
            for j in range(D // L):
                s = pl.ds(j * L, L)
                obr[0, i, s] = gbr[i, s] * SCALE

    for b in range(NBUF):
        gather_start(b, b)

    nblocks = rows_per_w // NBUF

    @pl.loop(0, nblocks)
    def _(blk):
        for b in range(NBUF):
            r = blk * NBUF + b
            gather_wait(r, b)

            @pl.when(blk > 0)
            def _():
                out_wait(r - NBUF, b)

            scale(b)
            out_start(r, b)

            @pl.when(blk < nblocks - 1)
            def _():
                gather_start(r + NBUF, b)

    for b in range(NBUF):
        out_wait((nblocks - 1) * NBUF + b, b)


@jax.jit
def kernel(x, table):
    Bx, Sx = x.shape
    x3 = x.reshape(Bx, Sx // G, G)
    mesh = plsc.VectorSubcoreMesh(
        core_axis_name="c", subcore_axis_name="s",
        num_cores=NC, num_subcores=NS,
    )
    rows_per_w = Bx // NW
    run = pl.kernel(
        _body,
        out_type=jax.ShapeDtypeStruct((Bx, Sx, D), jnp.float32),
        mesh=mesh,
        scratch_types=(
            [pltpu.VMEM((rows_per_w, S // G, G), jnp.int32)]
            + [pltpu.VMEM((S, D), jnp.float32) for _ in range(NBUF)]
            + [pltpu.VMEM((1, S, D), jnp.float32) for _ in range(NBUF)]
            + [pltpu.SemaphoreType.DMA for _ in range(2 * NBUF)]
        ),
        compiler_params=pltpu.CompilerParams(use_tc_tiling_on_sc=False),
    )
    return run(x3, table)


# speedup vs baseline: 1.0089x; 1.0089x over previous
import jax
import jax.numpy as jnp
from jax import lax
from jax.experimental import pallas as pl
from jax.experimental.pallas import tpu as pltpu
from jax.experimental.pallas import tpu_sc as plsc
from jax.experimental import layout as jex_layout

D = 64
L = 16
SCALE = 8.0
NC, NS = 2, 16
NW = NC * NS
C = 128
NBUF = 4


def _body(x_hbm, table_hbm, out_hbm, idx_all,
          gb0, gb1, gb2, gb3, ob0, ob1, ob2, ob3,
          gs0, gs1, gs2, gs3, os0, os1, os2, os3):
    gbuf = [gb0, gb1, gb2, gb3]
    obuf = [ob0, ob1, ob2, ob3]
    gsem = [gs0, gs1, gs2, gs3]
    osem = [os0, os1, os2, os3]

    wid = lax.axis_index("s") * NC + lax.axis_index("c")
    nchunks = out_hbm.shape[0] // NW
    nblocks = nchunks // NBUF
    cbase = wid * nchunks

    pltpu.sync_copy(x_hbm.at[pl.ds(cbase, nchunks)], idx_all)

    def gather_start(g, b):
        pltpu.async_copy(table_hbm.at[idx_all.at[g]], gbuf[b], gsem[b])

    def gather_wait(g, b):
        pltpu.make_async_copy(table_hbm.at[idx_all.at[g]], gbuf[b],
                              gsem[b]).wait()

    def out_start(g, b):
        pltpu.async_copy(obuf[b], out_hbm.at[pl.ds(cbase + g, 1)], osem[b])

    def out_wait(g, b):
        pltpu.make_async_copy(obuf[b], out_hbm.at[pl.ds(cbase + g, 1)],
                              osem[b]).wait()

    def scale(b):
        gbr, obr = gbuf[b], obuf[b]

        @pl.loop(0, C, unroll=4)
        def _(i):
            for j in range(D // L):
                s = pl.ds(j * L, L)
                obr[0, i, s] = gbr[i, s] * SCALE

    for b in range(NBUF):
        gather_start(b, b)

    @pl.loop(0, nblocks)
    def _(blk):
        for b in range(NBUF):
            g = blk * NBUF + b
            gather_wait(g, b)

            @pl.when(blk > 0)
            def _():
                out_wait(g - NBUF, b)

            scale(b)
            out_start(g, b)

            @pl.when(blk < nblocks - 1)
            def _():
                gather_start(g + NBUF, b)

    for b in range(NBUF):
        out_wait((nblocks - 1) * NBUF + b, b)


def _impl(x, table):
    Bx, Sx = x.shape
    B = Bx * Sx
    x2d = x.reshape(B // C, C)
    mesh = plsc.VectorSubcoreMesh(
        core_axis_name="c", subcore_axis_name="s",
        num_cores=NC, num_subcores=NS,
    )
    run = pl.kernel(
        _body,
        out_type=jax.ShapeDtypeStruct((B // C, C, D), jnp.float32),
        mesh=mesh,
        scratch_types=(
            [pltpu.VMEM((B // (NW * C), C), jnp.int32)]
            + [pltpu.VMEM((C, D), jnp.float32) for _ in range(NBUF)]
            + [pltpu.VMEM((1, C, D), jnp.float32) for _ in range(NBUF)]
            + [pltpu.SemaphoreType.DMA for _ in range(2 * NBUF)]
        ),
        compiler_params=pltpu.CompilerParams(use_tc_tiling_on_sc=False),
    )
    out = run(x2d, table)
    out = out.reshape(Bx, Sx, D)
    return out


def kernel(x, table):
    return _impl(x, table)


# speedup vs baseline: 1.1338x; 1.1238x over previous
import jax
import jax.numpy as jnp
from jax import lax
from jax.experimental import pallas as pl
from jax.experimental.pallas import tpu as pltpu
from jax.experimental.pallas import tpu_sc as plsc
from jax.experimental import layout as jex_layout

D = 64
L = 16
SCALE = 8.0
NC, NS = 2, 16
NW = NC * NS
C = 128
NBUF = 4


def _body(x_hbm, table_hbm, out_hbm, idx_all,
          gb0, gb1, gb2, gb3, ob0, ob1, ob2, ob3,
          gs0, gs1, gs2, gs3, os0, os1, os2, os3):
    gbuf = [gb0, gb1, gb2, gb3]
    obuf = [ob0, ob1, ob2, ob3]
    gsem = [gs0, gs1, gs2, gs3]
    osem = [os0, os1, os2, os3]

    wid = lax.axis_index("s") * NC + lax.axis_index("c")
    nchunks = out_hbm.shape[0] // NW
    nblocks = nchunks // NBUF
    cbase = wid * nchunks

    pltpu.sync_copy(x_hbm.at[pl.ds(cbase, nchunks)], idx_all)

    def gather_start(g, b):
        pltpu.async_copy(table_hbm.at[idx_all.at[g]], gbuf[b], gsem[b])

    def gather_wait(g, b):
        pltpu.make_async_copy(table_hbm.at[idx_all.at[g]], gbuf[b],
                              gsem[b]).wait()

    def out_start(g, b):
        pltpu.async_copy(obuf[b], out_hbm.at[pl.ds(cbase + g, 1)], osem[b])

    def out_wait(g, b):
        pltpu.make_async_copy(obuf[b], out_hbm.at[pl.ds(cbase + g, 1)],
                              osem[b]).wait()

    def scale(b):
        gbr, obr = gbuf[b], obuf[b]

        @pl.loop(0, C, unroll=4)
        def _(i):
            for j in range(D // L):
                s = pl.ds(j * L, L)
                obr[0, i, s] = gbr[i, s] * SCALE

    for b in range(NBUF):
        gather_start(b, b)

    @pl.loop(0, nblocks)
    def _(blk):
        for b in range(NBUF):
            g = blk * NBUF + b
            gather_wait(g, b)

            @pl.when(blk > 0)
            def _():
                out_wait(g - NBUF, b)

            scale(b)
            out_start(g, b)

            @pl.when(blk < nblocks - 1)
            def _():
                gather_start(g + NBUF, b)

    for b in range(NBUF):
        out_wait((nblocks - 1) * NBUF + b, b)


def _impl(x, table):
    Bx, Sx = x.shape
    B = Bx * Sx
    x2d = x.reshape(B // C, C)
    mesh = plsc.VectorSubcoreMesh(
        core_axis_name="c", subcore_axis_name="s",
        num_cores=NC, num_subcores=NS,
    )
    run = pl.kernel(
        _body,
        out_type=jax.ShapeDtypeStruct((B // C, C, D), jnp.float32),
        mesh=mesh,
        scratch_types=(
            [pltpu.VMEM((B // (NW * C), C), jnp.int32)]
            + [pltpu.VMEM((C, D), jnp.float32) for _ in range(NBUF)]
            + [pltpu.VMEM((1, C, D), jnp.float32) for _ in range(NBUF)]
            + [pltpu.SemaphoreType.DMA for _ in range(2 * NBUF)]
        ),
        compiler_params=pltpu.CompilerParams(use_tc_tiling_on_sc=False),
    )
    out = run(x2d, table)
    out = out.reshape(Bx, Sx, D)
    out = jex_layout.with_layout_constraint(
        out, jex_layout.Layout(major_to_minor=(0, 1, 2)))
    return out


def kernel(x, table):
    return _impl(x, table)


# speedup vs baseline: 1.2780x; 1.1272x over previous
import jax
import jax.numpy as jnp
from jax import lax
from jax.experimental import pallas as pl
from jax.experimental.pallas import tpu as pltpu
from jax.experimental.pallas import tpu_sc as plsc
from jax.experimental import layout as jex_layout

D = 64
L = 16
SCALE = 8.0
NC, NS = 2, 16
NW = NC * NS
C = 128
NBUF = 4


def _body(x_hbm, table_hbm, out_hbm, idx_all,
          gb0, gb1, gb2, gb3, ob0, ob1, ob2, ob3,
          gs0, gs1, gs2, gs3, os0, os1, os2, os3):
    gbuf = [gb0, gb1, gb2, gb3]
    obuf = [ob0, ob1, ob2, ob3]
    gsem = [gs0, gs1, gs2, gs3]
    osem = [os0, os1, os2, os3]

    wid = lax.axis_index("s") * NC + lax.axis_index("c")
    nchunks = out_hbm.shape[0] // NW
    nblocks = nchunks // NBUF
    cbase = wid * nchunks

    pltpu.sync_copy(x_hbm.at[pl.ds(cbase, nchunks)], idx_all)

    def gather_start(g, b):
        pltpu.async_copy(table_hbm.at[idx_all.at[g]], gbuf[b], gsem[b])

    def gather_wait(g, b):
        pltpu.make_async_copy(table_hbm.at[idx_all.at[g]], gbuf[b],
                              gsem[b]).wait()

    def out_start(g, b):
        pltpu.async_copy(obuf[b], out_hbm.at[pl.ds(cbase + g, 1)], osem[b])

    def out_wait(g, b):
        pltpu.make_async_copy(obuf[b], out_hbm.at[pl.ds(cbase + g, 1)],
                              osem[b]).wait()

    def scale(b):
        gbr, obr = gbuf[b], obuf[b]

        @plsc.parallel_loop(0, C, unroll=8)
        def _(i):
            for j in range(D // L):
                s = pl.ds(j * L, L)
                obr[0, i, s] = gbr[i, s] * SCALE

    for b in range(NBUF):
        gather_start(b, b)

    @pl.loop(0, nblocks)
    def _(blk):
        for b in range(NBUF):
            g = blk * NBUF + b
            gather_wait(g, b)

            @pl.when(blk > 0)
            def _():
                out_wait(g - NBUF, b)

            scale(b)
            out_start(g, b)

            @pl.when(blk < nblocks - 1)
            def _():
                gather_start(g + NBUF, b)

    for b in range(NBUF):
        out_wait((nblocks - 1) * NBUF + b, b)


def _impl(x, table):
    Bx, Sx = x.shape
    B = Bx * Sx
    x2d = x.reshape(B // C, C)
    mesh = plsc.VectorSubcoreMesh(
        core_axis_name="c", subcore_axis_name="s",
        num_cores=NC, num_subcores=NS,
    )
    run = pl.kernel(
        _body,
        out_type=jax.ShapeDtypeStruct((B // C, C, D), jnp.float32),
        mesh=mesh,
        scratch_types=(
            [pltpu.VMEM((B // (NW * C), C), jnp.int32)]
            + [pltpu.VMEM((C, D), jnp.float32) for _ in range(NBUF)]
            + [pltpu.VMEM((1, C, D), jnp.float32) for _ in range(NBUF)]
            + [pltpu.SemaphoreType.DMA for _ in range(2 * NBUF)]
        ),
        compiler_params=pltpu.CompilerParams(use_tc_tiling_on_sc=False),
    )
    out = run(x2d, table)
    return out.reshape(Bx, Sx, D)


def kernel(x, table):
    return _impl(x, table)
